# baseline (speedup 1.0000x reference)
import functools

import jax
import jax.numpy as jnp
from jax import lax
from jax.experimental import pallas as pl
from jax.experimental.pallas import tpu as pltpu
from jax.experimental.pallas import tpu_sc as plsc

N = 10000
E = 320000
D = 128

NC = 2
NS = 16
L = 16
NW = NC * NS

EPT = E // NW
CH = 80
NCH = EPT // CH
PH = 5
NCHP = NCH // PH
NCHG = E // CH
N_PAD = 10240
RPT = N_PAD // NS

_MESH = plsc.VectorSubcoreMesh(
    core_axis_name="c", subcore_axis_name="s", num_cores=NC, num_subcores=NS)


def _worker(c, s):
  return c * NS + s


@functools.partial(
    pl.kernel,
    out_type=jax.ShapeDtypeStruct((NW, N), jnp.float32),
    mesh=_MESH,
    scratch_types=[
        pltpu.VMEM((EPT,), jnp.int32),
        pltpu.VMEM((N,), jnp.float32),
    ],
    compiler_params=pltpu.CompilerParams(needs_layout_passes=False),
)
def _deg_kernel(dst_hbm, zeros_hbm, out_hbm, dst_v, hist_v):
  c = lax.axis_index("c")
  s = lax.axis_index("s")
  w = _worker(c, s)
  pltpu.sync_copy(zeros_hbm, hist_v)
  pltpu.sync_copy(dst_hbm.at[pl.ds(w * EPT, EPT)], dst_v)
  ones = jnp.full((L,), 1.0, dtype=jnp.float32)

  def body(i, carry):
    idx = dst_v[pl.ds(i * L, L)]
    plsc.addupdate_scatter(hist_v, [idx], ones)
    return carry

  lax.fori_loop(0, EPT // L, body, 0, unroll=4)
  pltpu.sync_copy(hist_v, out_hbm.at[w])


@functools.partial(
    pl.kernel,
    out_type=jax.ShapeDtypeStruct((NC, N_PAD, D), jnp.float32),
    mesh=_MESH,
    scratch_types=[
        pltpu.VMEM((NCHP, CH), jnp.int32),
        pltpu.VMEM((NCHP, CH), jnp.int32),
        pltpu.VMEM((CH, D), jnp.float32),
        pltpu.VMEM((CH, D), jnp.float32),
        pltpu.VMEM_SHARED((N_PAD, D), jnp.float32),
        pltpu.SemaphoreType.DMA,
        pltpu.SemaphoreType.DMA,
    ],
)
def _agg_kernel(xw_hbm, src_hbm, dst_hbm, zeros_hbm, out_hbm,
                sidx, didx, rbuf0, rbuf1, acc_sh, gsem0, gsem1):
  c = lax.axis_index("c")
  s = lax.axis_index("s")
  w = _worker(c, s)

  pltpu.sync_copy(zeros_hbm, acc_sh.at[pl.ds(s * RPT, RPT)])
  plsc.subcore_barrier()

  def gissue(j, buf, sem):
    pltpu.async_copy(xw_hbm.at[pl.ds(j * CH, CH)], buf, sem)

  def gwait(j, buf, sem):
    pltpu.make_async_copy(xw_hbm.at[pl.ds(j * CH, CH)], buf, sem).wait()

  def scatter(j, buf):
    pltpu.sync_copy(buf, acc_sh.at[didx.at[j]], add=True)

  for ph in range(PH):
    pltpu.sync_copy(src_hbm.at[w, ph], sidx)
    pltpu.sync_copy(dst_hbm.at[w, ph], didx)

    gissue(0, rbuf0, gsem0)

    def body(i, carry):
      j = 2 * i
      gwait(j, rbuf0, gsem0)
      gissue(j + 1, rbuf1, gsem1)
      scatter(j, rbuf0)
      gwait(j + 1, rbuf1, gsem1)

      @pl.when(j + 2 < NCHP)
      def _():
        gissue(j + 2, rbuf0, gsem0)

      scatter(j + 1, rbuf1)
      return carry

    lax.fori_loop(0, (NCHP - 1) // 2, body, 0)
    gwait(NCHP - 1, rbuf0, gsem0)
    scatter(NCHP - 1, rbuf0)

  plsc.subcore_barrier()
  pltpu.sync_copy(acc_sh.at[pl.ds(s * RPT, RPT)],
                  out_hbm.at[c, pl.ds(s * RPT, RPT)])


_BLK = 1000
_GRID = N // _BLK


def _row_spec(blk=_BLK, width=D):
  return pl.BlockSpec((blk, width), lambda i: (i, 0))


def _full_spec(a, b):
  return pl.BlockSpec((a, b), lambda i: (0, 0))


def _pspec():
  return pl.BlockSpec((NC, _BLK, D), lambda i: (0, i, 0))


def _ln(x, g, b):
  mu = jnp.mean(x, axis=-1, keepdims=True)
  xc = x - mu
  var = jnp.mean(xc * xc, axis=-1, keepdims=True)
  return xc * lax.rsqrt(var + 1e-5) * g + b


def _dinv_body(degp_ref, out_ref):
  deg = jnp.sum(degp_ref[...], axis=0, keepdims=True) + 1.0
  out_ref[...] = lax.rsqrt(deg).T


_dinv_kernel = pl.pallas_call(
    _dinv_body,
    grid=(79,),
    in_specs=[pl.BlockSpec((NW, 128), lambda i: (0, i))],
    out_specs=pl.BlockSpec((128, 1), lambda i: (i, 0)),
    out_shape=jax.ShapeDtypeStruct((N, 1), jnp.float32),
)


def _pre0_body(x_ref, w_ref, dinv_ref, out_ref):
  xw = jnp.dot(x_ref[...], w_ref[...], preferred_element_type=jnp.float32)
  out_ref[...] = dinv_ref[...] * xw


_pre0_kernel = pl.pallas_call(
    _pre0_body,
    grid=(_GRID,),
    in_specs=[_row_spec(), _full_spec(D, D), _row_spec(_BLK, 1)],
    out_specs=_row_spec(),
    out_shape=jax.ShapeDtypeStruct((N, D), jnp.float32),
)


def _mid_body(has_residual, p_ref, xw_ref, h_ref, dinv_ref, b_ref,
              g_ref, bb_ref, w_ref, h_out, xw_out):
  dinv = dinv_ref[...]
  gcn = dinv * (p_ref[0] + p_ref[1] + xw_ref[...]) + b_ref[...]
  h = jnp.maximum(gcn, 0.0)
  if has_residual:
    h = h + h_ref[...]
  h_out[...] = h
  t = _ln(h, g_ref[...], bb_ref[...])
  xw_out[...] = dinv * jnp.dot(t, w_ref[...],
                               preferred_element_type=jnp.float32)


def _make_mid(has_residual):
  return pl.pallas_call(
      functools.partial(_mid_body, has_residual),
      grid=(_GRID,),
      in_specs=[_pspec(), _row_spec(), _row_spec(), _row_spec(_BLK, 1),
                _full_spec(1, D), _full_spec(1, D), _full_spec(1, D),
                _full_spec(D, D)],
      out_specs=[_row_spec(), _row_spec()],
      out_shape=[jax.ShapeDtypeStruct((N, D), jnp.float32),
                 jax.ShapeDtypeStruct((N, D), jnp.float32)],
  )


_mid_kernel_res = _make_mid(True)
_mid_kernel_nores = _make_mid(False)


def _pre5_body(p_ref, xw_ref, h_ref, dinv_ref, b_ref, w_ref, xw_out):
  dinv = dinv_ref[...]
  gcn = dinv * (p_ref[0] + p_ref[1] + xw_ref[...]) + b_ref[...]
  h = jnp.maximum(gcn, 0.0) + h_ref[...]
  xw_out[...] = dinv * jnp.dot(h, w_ref[...],
                               preferred_element_type=jnp.float32)


_pre5_kernel = pl.pallas_call(
    _pre5_body,
    grid=(_GRID,),
    in_specs=[_pspec(), _row_spec(), _row_spec(), _row_spec(_BLK, 1),
              _full_spec(1, D), _full_spec(D, D)],
    out_specs=_row_spec(),
    out_shape=jax.ShapeDtypeStruct((N, D), jnp.float32),
)


def _final_body(p_ref, xw_ref, dinv_ref, b_ref, g_ref, bb_ref, out_ref):
  gcn = dinv_ref[...] * (p_ref[0] + p_ref[1] + xw_ref[...]) + b_ref[...]
  out_ref[...] = _ln(gcn, g_ref[...], bb_ref[...])


_final_kernel = pl.pallas_call(
    _final_body,
    grid=(_GRID,),
    in_specs=[_pspec(), _row_spec(), _row_spec(_BLK, 1),
              _full_spec(1, D), _full_spec(1, D), _full_spec(1, D)],
    out_specs=_row_spec(),
    out_shape=jax.ShapeDtypeStruct((N, D), jnp.float32),
)


def kernel(x, edge_index, W0, W1, W2, W3, W4, W5, b0, b1, b2, b3, b4, b5,
           ln_g1, ln_g2, ln_g3, ln_g4, ln_b1, ln_b2, ln_b3, ln_b4,
           fn_g, fn_b):
  src = edge_index[0].reshape(NW, PH, NCHP, CH)
  dst = edge_index[1].reshape(NW, PH, NCHP, CH)
  dst_flat = edge_index[1]
  zeros_n = jnp.zeros((N,), jnp.float32)
  zeros_nd = jnp.zeros((RPT, D), jnp.float32)

  degp = _deg_kernel(dst_flat, zeros_n)
  dinv = _dinv_kernel(degp)

  def r2(v):
    return v.reshape(1, D)

  xw = _pre0_kernel(x, W0, dinv)
  p = _agg_kernel(xw, src, dst, zeros_nd)
  h, xw = _mid_kernel_nores(p, xw, xw, dinv, r2(b0), r2(ln_g1), r2(ln_b1), W1)

  for (bi, g, bb, W) in ((b1, ln_g2, ln_b2, W2), (b2, ln_g3, ln_b3, W3),
                         (b3, ln_g4, ln_b4, W4)):
    p = _agg_kernel(xw, src, dst, zeros_nd)
    h, xw = _mid_kernel_res(p, xw, h, dinv, r2(bi), r2(g), r2(bb), W)

  p = _agg_kernel(xw, src, dst, zeros_nd)
  xw = _pre5_kernel(p, xw, h, dinv, r2(b4), W5)

  p = _agg_kernel(xw, src, dst, zeros_nd)
  return _final_kernel(p, xw, dinv, r2(b5), r2(fn_g), r2(fn_b))


# speedup vs baseline: 1.3810x; 1.3810x over previous
import functools

import jax
import jax.numpy as jnp
from jax import lax
from jax.experimental import pallas as pl
from jax.experimental.pallas import tpu as pltpu
from jax.experimental.pallas import tpu_sc as plsc

N = 10000
E = 320000
D = 128

NC = 2
NS = 16
L = 16
NW = NC * NS

EPT = E // NW
CH = 80
NCH = EPT // CH
PH = 5
NCHP = NCH // PH
NCHG = E // CH
N_PAD = 10240
RPT = N_PAD // NS

_MESH = plsc.VectorSubcoreMesh(
    core_axis_name="c", subcore_axis_name="s", num_cores=NC, num_subcores=NS)


def _worker(c, s):
  return c * NS + s


@functools.partial(
    pl.kernel,
    out_type=jax.ShapeDtypeStruct((NW, N), jnp.float32),
    mesh=_MESH,
    scratch_types=[
        pltpu.VMEM((EPT,), jnp.int32),
        pltpu.VMEM((N,), jnp.float32),
    ],
    compiler_params=pltpu.CompilerParams(needs_layout_passes=False),
)
def _deg_kernel(dst_hbm, zeros_hbm, out_hbm, dst_v, hist_v):
  c = lax.axis_index("c")
  s = lax.axis_index("s")
  w = _worker(c, s)
  pltpu.sync_copy(zeros_hbm, hist_v)
  pltpu.sync_copy(dst_hbm.at[pl.ds(w * EPT, EPT)], dst_v)
  ones = jnp.full((L,), 1.0, dtype=jnp.float32)

  def body(i, carry):
    idx = dst_v[pl.ds(i * L, L)]
    plsc.addupdate_scatter(hist_v, [idx], ones)
    return carry

  lax.fori_loop(0, EPT // L, body, 0, unroll=4)
  pltpu.sync_copy(hist_v, out_hbm.at[w])


@functools.partial(
    pl.kernel,
    out_type=jax.ShapeDtypeStruct((NC, N_PAD, D), jnp.float32),
    mesh=_MESH,
    scratch_types=[
        pltpu.VMEM((NCHP, CH), jnp.int32),
        pltpu.VMEM((NCHP, CH), jnp.int32),
        pltpu.VMEM((CH, D), jnp.float32),
        pltpu.VMEM((CH, D), jnp.float32),
        pltpu.VMEM((CH, D), jnp.float32),
        pltpu.VMEM_SHARED((N_PAD, D), jnp.float32),
        pltpu.SemaphoreType.DMA,
        pltpu.SemaphoreType.DMA,
        pltpu.SemaphoreType.DMA,
    ],
)
def _agg_kernel(xw_hbm, src_hbm, dst_hbm, zeros_hbm, out_hbm,
                sidx, didx, rbuf0, rbuf1, rbuf2, acc_sh, gsem0, gsem1, gsem2):
  c = lax.axis_index("c")
  s = lax.axis_index("s")
  w = _worker(c, s)

  pltpu.sync_copy(zeros_hbm, acc_sh.at[pl.ds(s * RPT, RPT)])
  plsc.subcore_barrier()

  def gissue(j, buf, sem):
    pltpu.async_copy(xw_hbm.at[sidx.at[j]], buf, sem)

  def gwait(j, buf, sem):
    pltpu.make_async_copy(xw_hbm.at[sidx.at[j]], buf, sem).wait()

  def scatter(j, buf):
    pltpu.sync_copy(buf, acc_sh.at[didx.at[j]], add=True)

  bufs = (rbuf0, rbuf1, rbuf2)
  sems = (gsem0, gsem1, gsem2)

  for ph in range(PH):
    pltpu.sync_copy(src_hbm.at[w, ph], sidx)
    pltpu.sync_copy(dst_hbm.at[w, ph], didx)

    gissue(0, rbuf0, gsem0)
    gissue(1, rbuf1, gsem1)

    def body(g, carry):
      j = 3 * g
      for k in range(3):
        gwait(j + k, bufs[k], sems[k])

        @pl.when(j + k + 2 < NCHP)
        def _():
          gissue(j + k + 2, bufs[(k + 2) % 3], sems[(k + 2) % 3])

        scatter(j + k, bufs[k])
      return carry

    lax.fori_loop(0, NCHP // 3, body, 0)
    gwait(NCHP - 1, rbuf0, gsem0)
    scatter(NCHP - 1, rbuf0)

  plsc.subcore_barrier()
  pltpu.sync_copy(acc_sh.at[pl.ds(s * RPT, RPT)],
                  out_hbm.at[c, pl.ds(s * RPT, RPT)])


_BLK = 1000
_GRID = N // _BLK


def _row_spec(blk=_BLK, width=D):
  return pl.BlockSpec((blk, width), lambda i: (i, 0))


def _full_spec(a, b):
  return pl.BlockSpec((a, b), lambda i: (0, 0))


def _pspec():
  return pl.BlockSpec((NC, _BLK, D), lambda i: (0, i, 0))


def _ln(x, g, b):
  mu = jnp.mean(x, axis=-1, keepdims=True)
  xc = x - mu
  var = jnp.mean(xc * xc, axis=-1, keepdims=True)
  return xc * lax.rsqrt(var + 1e-5) * g + b


def _dinv_body(degp_ref, out_ref):
  deg = jnp.sum(degp_ref[...], axis=0, keepdims=True) + 1.0
  out_ref[...] = lax.rsqrt(deg).T


_dinv_kernel = pl.pallas_call(
    _dinv_body,
    grid=(79,),
    in_specs=[pl.BlockSpec((NW, 128), lambda i: (0, i))],
    out_specs=pl.BlockSpec((128, 1), lambda i: (i, 0)),
    out_shape=jax.ShapeDtypeStruct((N, 1), jnp.float32),
)


def _pre0_body(x_ref, w_ref, dinv_ref, out_ref):
  xw = jnp.dot(x_ref[...], w_ref[...], preferred_element_type=jnp.float32)
  out_ref[...] = dinv_ref[...] * xw


_pre0_kernel = pl.pallas_call(
    _pre0_body,
    grid=(_GRID,),
    in_specs=[_row_spec(), _full_spec(D, D), _row_spec(_BLK, 1)],
    out_specs=_row_spec(),
    out_shape=jax.ShapeDtypeStruct((N, D), jnp.float32),
)


def _mid_body(has_residual, p_ref, xw_ref, h_ref, dinv_ref, b_ref,
              g_ref, bb_ref, w_ref, h_out, xw_out):
  dinv = dinv_ref[...]
  gcn = dinv * (p_ref[0] + p_ref[1] + xw_ref[...]) + b_ref[...]
  h = jnp.maximum(gcn, 0.0)
  if has_residual:
    h = h + h_ref[...]
  h_out[...] = h
  t = _ln(h, g_ref[...], bb_ref[...])
  xw_out[...] = dinv * jnp.dot(t, w_ref[...],
                               preferred_element_type=jnp.float32)


def _make_mid(has_residual):
  return pl.pallas_call(
      functools.partial(_mid_body, has_residual),
      grid=(_GRID,),
      in_specs=[_pspec(), _row_spec(), _row_spec(), _row_spec(_BLK, 1),
                _full_spec(1, D), _full_spec(1, D), _full_spec(1, D),
                _full_spec(D, D)],
      out_specs=[_row_spec(), _row_spec()],
      out_shape=[jax.ShapeDtypeStruct((N, D), jnp.float32),
                 jax.ShapeDtypeStruct((N, D), jnp.float32)],
  )


_mid_kernel_res = _make_mid(True)
_mid_kernel_nores = _make_mid(False)


def _pre5_body(p_ref, xw_ref, h_ref, dinv_ref, b_ref, w_ref, xw_out):
  dinv = dinv_ref[...]
  gcn = dinv * (p_ref[0] + p_ref[1] + xw_ref[...]) + b_ref[...]
  h = jnp.maximum(gcn, 0.0) + h_ref[...]
  xw_out[...] = dinv * jnp.dot(h, w_ref[...],
                               preferred_element_type=jnp.float32)


_pre5_kernel = pl.pallas_call(
    _pre5_body,
    grid=(_GRID,),
    in_specs=[_pspec(), _row_spec(), _row_spec(), _row_spec(_BLK, 1),
              _full_spec(1, D), _full_spec(D, D)],
    out_specs=_row_spec(),
    out_shape=jax.ShapeDtypeStruct((N, D), jnp.float32),
)


def _final_body(p_ref, xw_ref, dinv_ref, b_ref, g_ref, bb_ref, out_ref):
  gcn = dinv_ref[...] * (p_ref[0] + p_ref[1] + xw_ref[...]) + b_ref[...]
  out_ref[...] = _ln(gcn, g_ref[...], bb_ref[...])


_final_kernel = pl.pallas_call(
    _final_body,
    grid=(_GRID,),
    in_specs=[_pspec(), _row_spec(), _row_spec(_BLK, 1),
              _full_spec(1, D), _full_spec(1, D), _full_spec(1, D)],
    out_specs=_row_spec(),
    out_shape=jax.ShapeDtypeStruct((N, D), jnp.float32),
)


def kernel(x, edge_index, W0, W1, W2, W3, W4, W5, b0, b1, b2, b3, b4, b5,
           ln_g1, ln_g2, ln_g3, ln_g4, ln_b1, ln_b2, ln_b3, ln_b4,
           fn_g, fn_b):
  src = edge_index[0].reshape(NW, PH, NCHP, CH)
  dst = edge_index[1].reshape(NW, PH, NCHP, CH)
  dst_flat = edge_index[1]
  zeros_n = jnp.zeros((N,), jnp.float32)
  zeros_nd = jnp.zeros((RPT, D), jnp.float32)

  degp = _deg_kernel(dst_flat, zeros_n)
  dinv = _dinv_kernel(degp)

  def r2(v):
    return v.reshape(1, D)

  xw = _pre0_kernel(x, W0, dinv)
  p = _agg_kernel(xw, src, dst, zeros_nd)
  h, xw = _mid_kernel_nores(p, xw, xw, dinv, r2(b0), r2(ln_g1), r2(ln_b1), W1)

  for (bi, g, bb, W) in ((b1, ln_g2, ln_b2, W2), (b2, ln_g3, ln_b3, W3),
                         (b3, ln_g4, ln_b4, W4)):
    p = _agg_kernel(xw, src, dst, zeros_nd)
    h, xw = _mid_kernel_res(p, xw, h, dinv, r2(bi), r2(g), r2(bb), W)

  p = _agg_kernel(xw, src, dst, zeros_nd)
  xw = _pre5_kernel(p, xw, h, dinv, r2(b4), W5)

  p = _agg_kernel(xw, src, dst, zeros_nd)
  return _final_kernel(p, xw, dinv, r2(b5), r2(fn_g), r2(fn_b))


# speedup vs baseline: 1.4304x; 1.0358x over previous
import functools

import jax
import jax.numpy as jnp
from jax import lax
from jax.experimental import pallas as pl
from jax.experimental.pallas import tpu as pltpu
from jax.experimental.pallas import tpu_sc as plsc

N = 10000
E = 320000
D = 128

NC = 2
NS = 16
L = 16
NW = NC * NS

EPT = E // NW
CH = 80
NCH = EPT // CH
PH = 5
NCHP = NCH // PH
NCHG = E // CH
N_PAD = 10112
RPT = N_PAD // NS

_MESH = plsc.VectorSubcoreMesh(
    core_axis_name="c", subcore_axis_name="s", num_cores=NC, num_subcores=NS)


def _worker(c, s):
  return c * NS + s


@functools.partial(
    pl.kernel,
    out_type=jax.ShapeDtypeStruct((NW, N), jnp.float32),
    mesh=_MESH,
    scratch_types=[
        pltpu.VMEM((EPT,), jnp.int32),
        pltpu.VMEM((N,), jnp.float32),
    ],
    compiler_params=pltpu.CompilerParams(needs_layout_passes=False),
)
def _deg_kernel(dst_hbm, zeros_hbm, out_hbm, dst_v, hist_v):
  c = lax.axis_index("c")
  s = lax.axis_index("s")
  w = _worker(c, s)
  pltpu.sync_copy(zeros_hbm, hist_v)
  pltpu.sync_copy(dst_hbm.at[pl.ds(w * EPT, EPT)], dst_v)
  ones = jnp.full((L,), 1.0, dtype=jnp.float32)

  def body(i, carry):
    idx = dst_v[pl.ds(i * L, L)]
    plsc.addupdate_scatter(hist_v, [idx], ones)
    return carry

  lax.fori_loop(0, EPT // L, body, 0, unroll=4)
  pltpu.sync_copy(hist_v, out_hbm.at[w])


@functools.partial(
    pl.kernel,
    out_type=jax.ShapeDtypeStruct((NC, N_PAD, D), jnp.float32),
    mesh=_MESH,
    scratch_types=[
        pltpu.VMEM((NCHP, CH), jnp.int32),
        pltpu.VMEM((NCHP, CH), jnp.int32),
        pltpu.VMEM((CH, D), jnp.float32),
        pltpu.VMEM((CH, D), jnp.float32),
        pltpu.VMEM((CH, D), jnp.float32),
        pltpu.VMEM((CH, D), jnp.float32),
        pltpu.VMEM_SHARED((N_PAD, D), jnp.float32),
        pltpu.SemaphoreType.DMA,
        pltpu.SemaphoreType.DMA,
        pltpu.SemaphoreType.DMA,
        pltpu.SemaphoreType.DMA,
    ],
)
def _agg_kernel(xw_hbm, src_hbm, dst_hbm, zeros_hbm, out_hbm,
                sidx, didx, rbuf0, rbuf1, rbuf2, rbuf3, acc_sh,
                gsem0, gsem1, gsem2, gsem3):
  c = lax.axis_index("c")
  s = lax.axis_index("s")
  w = _worker(c, s)

  pltpu.sync_copy(zeros_hbm, acc_sh.at[pl.ds(s * RPT, RPT)])
  plsc.subcore_barrier()

  def gissue(j, buf, sem):
    pltpu.async_copy(xw_hbm.at[sidx.at[j]], buf, sem)

  def gwait(j, buf, sem):
    pltpu.make_async_copy(xw_hbm.at[sidx.at[j]], buf, sem).wait()

  def scatter(j, buf):
    pltpu.sync_copy(buf, acc_sh.at[didx.at[j]], add=True)

  bufs = (rbuf0, rbuf1, rbuf2, rbuf3)
  sems = (gsem0, gsem1, gsem2, gsem3)
  NB = 4

  for ph in range(PH):
    pltpu.sync_copy(src_hbm.at[w, ph], sidx)
    pltpu.sync_copy(dst_hbm.at[w, ph], didx)

    for k in range(NB - 1):
      gissue(k, bufs[k], sems[k])

    def body(g, carry):
      j = NB * g
      for k in range(NB):
        gwait(j + k, bufs[k % NB], sems[k % NB])

        @pl.when(j + k + NB - 1 < NCHP)
        def _():
          gissue(j + k + NB - 1, bufs[(k + NB - 1) % NB], sems[(k + NB - 1) % NB])

        scatter(j + k, bufs[k % NB])
      return carry

    lax.fori_loop(0, NCHP // NB, body, 0)
    gwait(NCHP - 1, rbuf0, gsem0)
    scatter(NCHP - 1, rbuf0)

  plsc.subcore_barrier()
  pltpu.sync_copy(acc_sh.at[pl.ds(s * RPT, RPT)],
                  out_hbm.at[c, pl.ds(s * RPT, RPT)])


_BLK = 1000
_GRID = N // _BLK


def _row_spec(blk=_BLK, width=D):
  return pl.BlockSpec((blk, width), lambda i: (i, 0))


def _full_spec(a, b):
  return pl.BlockSpec((a, b), lambda i: (0, 0))


def _pspec():
  return pl.BlockSpec((NC, _BLK, D), lambda i: (0, i, 0))


def _ln(x, g, b):
  mu = jnp.mean(x, axis=-1, keepdims=True)
  xc = x - mu
  var = jnp.mean(xc * xc, axis=-1, keepdims=True)
  return xc * lax.rsqrt(var + 1e-5) * g + b


def _dinv_body(degp_ref, out_ref):
  deg = jnp.sum(degp_ref[...], axis=0, keepdims=True) + 1.0
  out_ref[...] = lax.rsqrt(deg).T


_dinv_kernel = pl.pallas_call(
    _dinv_body,
    grid=(79,),
    in_specs=[pl.BlockSpec((NW, 128), lambda i: (0, i))],
    out_specs=pl.BlockSpec((128, 1), lambda i: (i, 0)),
    out_shape=jax.ShapeDtypeStruct((N, 1), jnp.float32),
)


def _pre0_body(x_ref, w_ref, dinv_ref, out_ref):
  xw = jnp.dot(x_ref[...], w_ref[...], preferred_element_type=jnp.float32)
  out_ref[...] = dinv_ref[...] * xw


_pre0_kernel = pl.pallas_call(
    _pre0_body,
    grid=(_GRID,),
    in_specs=[_row_spec(), _full_spec(D, D), _row_spec(_BLK, 1)],
    out_specs=_row_spec(),
    out_shape=jax.ShapeDtypeStruct((N, D), jnp.float32),
)


def _mid_body(has_residual, p_ref, xw_ref, h_ref, dinv_ref, b_ref,
              g_ref, bb_ref, w_ref, h_out, xw_out):
  dinv = dinv_ref[...]
  gcn = dinv * (p_ref[0] + p_ref[1] + xw_ref[...]) + b_ref[...]
  h = jnp.maximum(gcn, 0.0)
  if has_residual:
    h = h + h_ref[...]
  h_out[...] = h
  t = _ln(h, g_ref[...], bb_ref[...])
  xw_out[...] = dinv * jnp.dot(t, w_ref[...],
                               preferred_element_type=jnp.float32)


def _make_mid(has_residual):
  return pl.pallas_call(
      functools.partial(_mid_body, has_residual),
      grid=(_GRID,),
      in_specs=[_pspec(), _row_spec(), _row_spec(), _row_spec(_BLK, 1),
                _full_spec(1, D), _full_spec(1, D), _full_spec(1, D),
                _full_spec(D, D)],
      out_specs=[_row_spec(), _row_spec()],
      out_shape=[jax.ShapeDtypeStruct((N, D), jnp.float32),
                 jax.ShapeDtypeStruct((N, D), jnp.float32)],
  )


_mid_kernel_res = _make_mid(True)
_mid_kernel_nores = _make_mid(False)


def _pre5_body(p_ref, xw_ref, h_ref, dinv_ref, b_ref, w_ref, xw_out):
  dinv = dinv_ref[...]
  gcn = dinv * (p_ref[0] + p_ref[1] + xw_ref[...]) + b_ref[...]
  h = jnp.maximum(gcn, 0.0) + h_ref[...]
  xw_out[...] = dinv * jnp.dot(h, w_ref[...],
                               preferred_element_type=jnp.float32)


_pre5_kernel = pl.pallas_call(
    _pre5_body,
    grid=(_GRID,),
    in_specs=[_pspec(), _row_spec(), _row_spec(), _row_spec(_BLK, 1),
              _full_spec(1, D), _full_spec(D, D)],
    out_specs=_row_spec(),
    out_shape=jax.ShapeDtypeStruct((N, D), jnp.float32),
)


def _final_body(p_ref, xw_ref, dinv_ref, b_ref, g_ref, bb_ref, out_ref):
  gcn = dinv_ref[...] * (p_ref[0] + p_ref[1] + xw_ref[...]) + b_ref[...]
  out_ref[...] = _ln(gcn, g_ref[...], bb_ref[...])


_final_kernel = pl.pallas_call(
    _final_body,
    grid=(_GRID,),
    in_specs=[_pspec(), _row_spec(), _row_spec(_BLK, 1),
              _full_spec(1, D), _full_spec(1, D), _full_spec(1, D)],
    out_specs=_row_spec(),
    out_shape=jax.ShapeDtypeStruct((N, D), jnp.float32),
)


def kernel(x, edge_index, W0, W1, W2, W3, W4, W5, b0, b1, b2, b3, b4, b5,
           ln_g1, ln_g2, ln_g3, ln_g4, ln_b1, ln_b2, ln_b3, ln_b4,
           fn_g, fn_b):
  src = edge_index[0].reshape(NW, PH, NCHP, CH)
  dst = edge_index[1].reshape(NW, PH, NCHP, CH)
  dst_flat = edge_index[1]
  zeros_n = jnp.zeros((N,), jnp.float32)
  zeros_nd = jnp.zeros((RPT, D), jnp.float32)

  degp = _deg_kernel(dst_flat, zeros_n)
  dinv = _dinv_kernel(degp)

  def r2(v):
    return v.reshape(1, D)

  xw = _pre0_kernel(x, W0, dinv)
  p = _agg_kernel(xw, src, dst, zeros_nd)
  h, xw = _mid_kernel_nores(p, xw, xw, dinv, r2(b0), r2(ln_g1), r2(ln_b1), W1)

  for (bi, g, bb, W) in ((b1, ln_g2, ln_b2, W2), (b2, ln_g3, ln_b3, W3),
                         (b3, ln_g4, ln_b4, W4)):
    p = _agg_kernel(xw, src, dst, zeros_nd)
    h, xw = _mid_kernel_res(p, xw, h, dinv, r2(bi), r2(g), r2(bb), W)

  p = _agg_kernel(xw, src, dst, zeros_nd)
  xw = _pre5_kernel(p, xw, h, dinv, r2(b4), W5)

  p = _agg_kernel(xw, src, dst, zeros_nd)
  return _final_kernel(p, xw, dinv, r2(b5), r2(fn_g), r2(fn_b))
